# baseline (speedup 1.0000x reference)
import jax
import jax.numpy as jnp
from jax import lax
from jax.experimental import pallas as pl
from jax.experimental.pallas import tpu as pltpu
from jax.experimental.pallas import tpu_sc as plsc

_B, _N, _Q, _K = 512, 1024, 21, 64
_NC, _NS = 2, 16
_NW = _NC * _NS
_BPW = _B // _NW
_RPW = _BPW * _Q
_NV = _N // 16
_ROWW = 3 * _N
_CHW = 8 * _ROWW
_TPW = _BPW * _Q * 3
_SELR = 168
_SELCAP = 144


def _sc_body(pc_hbm, tgt_hbm, out_hbm, pcf_v, tgt_v, x6_v, y6_v, z6_v,
             npc_v, d_v, idx_v, cva, cia, cvb, cib, sel_v, ob_a, ob_b,
             cnt_s, sem_a, sem_b):
    wid = lax.axis_index("s") * _NC + lax.axis_index("c")
    i16 = lax.iota(jnp.int32, 16)
    zf = jnp.zeros((16,), jnp.float32)
    zi = jnp.zeros((16,), jnp.int32)

    def pcnt(m):
        return jnp.max(plsc.all_reduce_population_count(m))

    def bround(x):
        u = lax.bitcast_convert_type(x, jnp.int32)
        u = (u + 32767 + ((u >> 16) & 1)) & jnp.int32(-65536)
        return lax.bitcast_convert_type(u, jnp.float32)

    pltpu.sync_copy(tgt_hbm.at[pl.ds(wid * _TPW, _TPW)], tgt_v)

    def init_i(i, _):
        idx_v[pl.ds(i * 16, 16)] = i16 + i * 16
        return 0
    lax.fori_loop(0, _NV, init_i, 0)

    def init_s(i, _):
        sel_v[pl.ds(i * 16, 16)] = zi
        return 0
    lax.fori_loop(0, (16 * _SELR) // 16, init_s, 0)

    def init_o(i, _):
        ob_a[pl.ds(i * 16, 16)] = zf
        ob_b[pl.ds(i * 16, 16)] = zf
        return 0
    lax.fori_loop(0, _CHW // 16, init_o, 0)

    def fill_row(r, slot, par, obuf):
        bl = r // _Q
        q = r - bl * _Q
        b = wid * _BPW + bl
        selbase = par * (8 * _SELR) + slot * _SELR

        @pl.when(q == 0)
        def _():
            pltpu.sync_copy(pc_hbm.at[pl.ds(b * _ROWW, _ROWW)], pcf_v)

            def pr(i, _2):
                o = i * 16
                i3 = (i16 + o) * 3
                x = plsc.load_gather(pcf_v, [i3])
                y = plsc.load_gather(pcf_v, [i3 + 1])
                z = plsc.load_gather(pcf_v, [i3 + 2])
                npc_v[pl.ds(o, 16)] = (x * x + y * y) + z * z
                x6_v[pl.ds(o, 16)] = bround(x)
                y6_v[pl.ds(o, 16)] = bround(y)
                z6_v[pl.ds(o, 16)] = bround(z)
                return 0
            lax.fori_loop(0, _NV, pr, 0)

        to = jnp.full((16,), 3 * r, jnp.int32)
        txr = plsc.load_gather(tgt_v, [to])
        tyr = plsc.load_gather(tgt_v, [to + 1])
        tzr = plsc.load_gather(tgt_v, [to + 2])
        nt = (txr * txr + tyr * tyr) + tzr * tzr
        tx6 = bround(txr)
        ty6 = bround(tyr)
        tz6 = bround(tzr)
        tx2, ty2, tz2 = tx6 + tx6, ty6 + ty6, tz6 + tz6

        def p0(i, _2):
            o = i * 16
            x = x6_v[pl.ds(o, 16)]
            y = y6_v[pl.ds(o, 16)]
            z = z6_v[pl.ds(o, 16)]
            npc = npc_v[pl.ds(o, 16)]
            dot2 = (tx2 * x + ty2 * y) + tz2 * z
            d_v[pl.ds(o, 16)] = (nt + npc) - dot2
            return 0
        lax.fori_loop(0, _NV, p0, 0)

        samp = plsc.load_gather(d_v, [i16 * _NV])
        ssort = jnp.sort(samp)
        pv = jnp.full((16,), jnp.max(jnp.where(i16 == 2, ssort,
                                               -jnp.inf)), jnp.float32)

        def cnt1(i, st):
            cbv, cev = st
            v = d_v[pl.ds(i * 16, 16)]
            cbv = cbv + plsc.all_reduce_population_count(v < pv)
            cev = cev + plsc.all_reduce_population_count(v == pv)
            return cbv, cev
        cbv, cev = lax.fori_loop(0, _NV, cnt1, (zi, zi))
        cb = jnp.max(cbv)

        def r1_below(_2):
            def cp(i, off):
                v = d_v[pl.ds(i * 16, 16)]
                ix = idx_v[pl.ds(i * 16, 16)]
                m = v < pv
                plsc.store_compressed(cva.at[pl.ds(off, 16)], v, mask=m)
                plsc.store_compressed(cia.at[pl.ds(off, 16)], ix, mask=m)
                return off + pcnt(m)
            lax.fori_loop(0, _NV, cp, jnp.int32(0))
            return jnp.int32(_K), cb, jnp.int32(0)

        def r1_above(_2):
            def cp(i, st):
                so, off = st
                v = d_v[pl.ds(i * 16, 16)]
                ix = idx_v[pl.ds(i * 16, 16)]
                mle = v <= pv
                mgt = v > pv
                sdst = selbase + jnp.minimum(so, _SELCAP)
                plsc.store_compressed(sel_v.at[pl.ds(sdst, 16)], ix, mask=mle)
                plsc.store_compressed(cva.at[pl.ds(off, 16)], v, mask=mgt)
                plsc.store_compressed(cia.at[pl.ds(off, 16)], ix, mask=mgt)
                return so + pcnt(mle), off + pcnt(mgt)
            so, off = lax.fori_loop(0, _NV, cp, (jnp.int32(0), jnp.int32(0)))
            return jnp.maximum(_K - so, 0), off, so

        r, c, soff = lax.cond(cb >= _K, r1_below, r1_above, 0)

        def wcond(st):
            r, c, par2, soff = st
            return jnp.logical_and(r > 0, c > 16)

        def wbody(st):
            r, c, par2, soff = st

            def rnd(sv, si, dv, di):
                ssort = jnp.sort(sv[pl.ds(0, 16)])
                j = jnp.clip((17 * r) // c, 0, 15)
                pv = jnp.full((16,), jnp.max(jnp.where(i16 == j, ssort,
                                                       -jnp.inf)), jnp.float32)
                nv = (c + 15) // 16

                def cnt2(i, st2):
                    cbv2, _unused = st2
                    v = sv[pl.ds(i * 16, 16)]
                    lm = (i16 + i * 16) < c
                    cbv2 = cbv2 + plsc.all_reduce_population_count(
                        (v < pv) & lm)
                    return cbv2, _unused
                cbv2, _3 = lax.fori_loop(0, nv, cnt2, (zi, zi))
                cb2 = jnp.max(cbv2)

                def below(_2):
                    def cp(i, off):
                        v = sv[pl.ds(i * 16, 16)]
                        ix = si[pl.ds(i * 16, 16)]
                        m = (v < pv) & ((i16 + i * 16) < c)
                        plsc.store_compressed(dv.at[pl.ds(off, 16)], v, mask=m)
                        plsc.store_compressed(di.at[pl.ds(off, 16)], ix,
                                              mask=m)
                        return off + pcnt(m)
                    lax.fori_loop(0, nv, cp, jnp.int32(0))
                    return r, cb2, soff

                def above(_2):
                    def cp(i, st3):
                        so, off = st3
                        v = sv[pl.ds(i * 16, 16)]
                        ix = si[pl.ds(i * 16, 16)]
                        lm = (i16 + i * 16) < c
                        mle = (v <= pv) & lm
                        mgt = (v > pv) & lm
                        sdst = selbase + jnp.minimum(so, _SELCAP)
                        plsc.store_compressed(sel_v.at[pl.ds(sdst, 16)], ix,
                                              mask=mle)
                        plsc.store_compressed(dv.at[pl.ds(off, 16)], v,
                                              mask=mgt)
                        plsc.store_compressed(di.at[pl.ds(off, 16)], ix,
                                              mask=mgt)
                        return so + pcnt(mle), off + pcnt(mgt)
                    so, off = lax.fori_loop(0, nv, cp, (soff, jnp.int32(0)))
                    return jnp.maximum(r - (so - soff), 0), off, so

                return lax.cond(cb2 >= r, below, above, 0)

            rr, cc, ss = lax.cond(
                par2 == 0,
                lambda _2: rnd(cva, cia, cvb, cib),
                lambda _2: rnd(cvb, cib, cva, cia), 0)
            return rr, cc, 1 - par2, ss

        r, c, par2, soff = lax.while_loop(
            wcond, wbody, (r, c, jnp.int32(0), soff))

        def final(_2):
            v, ix = lax.cond(
                par2 == 0,
                lambda _3: (cva[pl.ds(0, 16)], cia[pl.ds(0, 16)]),
                lambda _3: (cvb[pl.ds(0, 16)], cib[pl.ds(0, 16)]), 0)
            vk = jnp.where(i16 < c, v, jnp.inf)
            _4, sx = plsc.sort_key_val(vk, ix)
            sdst = selbase + jnp.minimum(soff, _SELCAP)
            plsc.store_compressed(sel_v.at[pl.ds(sdst, 16)], sx, mask=i16 < r)
            return soff + r
        stot = lax.cond(r > 0, final, lambda _2: soff, 0)
        stot = jnp.minimum(stot, _SELCAP + 16)
        cnt_s[par * 8 + slot] = stot

        def wr(u, _2):
            n16 = sel_v[pl.ds(selbase + u * 16, 16)]
            lm = (i16 + u * 16) < stot
            p0_ = n16 * 3
            po = slot * _ROWW + p0_
            for cc_ in range(3):
                val = plsc.load_gather(pcf_v, [p0_ + cc_], mask=lm)
                plsc.store_scatter(obuf, [po + cc_], val, mask=lm)
            return 0
        lax.fori_loop(0, (stot + 15) // 16, wr, 0)

    def wait_rezero(obuf, sem, par):
        pltpu.make_async_copy(out_hbm.at[pl.ds(0, _CHW)], obuf, sem).wait()

        def slotf(s, _):
            cnt = cnt_s[par * 8 + s]
            sb = par * (8 * _SELR) + s * _SELR

            def rz(u, _2):
                n16 = sel_v[pl.ds(sb + u * 16, 16)]
                lm = (i16 + u * 16) < cnt
                po = s * _ROWW + n16 * 3
                plsc.store_scatter(obuf, [po], zf, mask=lm)
                plsc.store_scatter(obuf, [po + 1], zf, mask=lm)
                plsc.store_scatter(obuf, [po + 2], zf, mask=lm)
                return 0
            lax.fori_loop(0, (cnt + 15) // 16, rz, 0)
            return 0
        lax.fori_loop(0, 8, slotf, 0)

    def half(j, par, obuf, sem, wait_first):
        if wait_first:
            wait_rezero(obuf, sem, par)

        def rowf(i, _):
            fill_row(16 * j + 8 * par + i, i, par, obuf)
            return 0
        lax.fori_loop(0, 8, rowf, 0)
        dst = (wid * _RPW + 16 * j + 8 * par) * _ROWW
        pltpu.async_copy(obuf, out_hbm.at[pl.ds(dst, _CHW)], sem)

    half(0, 0, ob_a, sem_a, False)
    half(0, 1, ob_b, sem_b, False)

    def superchunk(j, _):
        half(j, 0, ob_a, sem_a, True)
        half(j, 1, ob_b, sem_b, True)
        return 0
    lax.fori_loop(1, _Q, superchunk, 0)
    pltpu.make_async_copy(out_hbm.at[pl.ds(0, _CHW)], ob_a, sem_a).wait()
    pltpu.make_async_copy(out_hbm.at[pl.ds(0, _CHW)], ob_b, sem_b).wait()


@jax.jit
def kernel(pointcloud, target):
    pc = pointcloud[..., :3]
    b, n, _ = pc.shape
    q = target.shape[1]
    mesh = plsc.VectorSubcoreMesh(core_axis_name="c", subcore_axis_name="s")
    out = pl.kernel(
        _sc_body,
        out_type=jax.ShapeDtypeStruct((b * q * 3 * n,), jnp.float32),
        mesh=mesh,
        compiler_params=pltpu.CompilerParams(needs_layout_passes=False),
        scratch_types=[
            pltpu.VMEM((_ROWW,), jnp.float32),
            pltpu.VMEM((_TPW,), jnp.float32),
            pltpu.VMEM((n,), jnp.float32),
            pltpu.VMEM((n,), jnp.float32),
            pltpu.VMEM((n,), jnp.float32),
            pltpu.VMEM((n,), jnp.float32),
            pltpu.VMEM((n,), jnp.float32),
            pltpu.VMEM((n,), jnp.int32),
            pltpu.VMEM((1040,), jnp.float32),
            pltpu.VMEM((1040,), jnp.int32),
            pltpu.VMEM((1040,), jnp.float32),
            pltpu.VMEM((1040,), jnp.int32),
            pltpu.VMEM((16 * _SELR,), jnp.int32),
            pltpu.VMEM((_CHW,), jnp.float32),
            pltpu.VMEM((_CHW,), jnp.float32),
            pltpu.SMEM((16,), jnp.int32),
            pltpu.SemaphoreType.DMA,
            pltpu.SemaphoreType.DMA,
        ],
    )(pc.reshape(b * n * 3), target.reshape(b * q * 3))
    return out.reshape(b, q, n, 3)


# speedup vs baseline: 4.7149x; 4.7149x over previous
import jax
import jax.numpy as jnp
from jax.experimental import pallas as pl

_K = 64


def _body(pc_ref, tgt_ref, ox_ref, oy_ref, oz_ref):
    pc = pc_ref[0]
    tgt = tgt_ref[0]
    px = pc[0:1, :]
    py = pc[1:2, :]
    pz = pc[2:3, :]
    npc = px * px + py * py + pz * pz
    nt = jnp.sum(tgt * tgt, axis=1, keepdims=True)
    def r16(x):
        return x.astype(jnp.bfloat16).astype(jnp.float32)
    dot = (r16(tgt[:, 0:1]) * r16(px) + r16(tgt[:, 1:2]) * r16(py)
           + r16(tgt[:, 2:3]) * r16(pz))
    d = nt + npc - 2.0 * dot

    s = jax.lax.bitcast_convert_type(d, jnp.int32)
    int_min = jnp.int32(-(2**31))
    keys = jnp.where(s >= 0, s, jnp.bitwise_xor(jnp.bitwise_not(s), int_min))

    q = keys.shape[0]
    lo0 = jnp.full((q, 1), -(2**31), jnp.int32)
    hi0 = jnp.full((q, 1), 2**31 - 1, jnp.int32)
    tf0 = jnp.full((q, 1), 2**31 - 1, jnp.int32)
    fnd0 = jnp.zeros((q, 1), jnp.int32)

    def cond(st):
        it, lo, hi, tf, fnd = st
        return jnp.logical_and(it < 32, jnp.min(fnd) == 0)

    def step(st):
        it, lo, hi, tf, fnd = st
        mid = (lo >> 1) + (hi >> 1) + (lo & hi & 1)
        cnt = jnp.sum((keys <= mid).astype(jnp.int32), axis=1, keepdims=True)
        hit = jnp.logical_and(cnt == _K, fnd == 0)
        tf = jnp.where(hit, mid, tf)
        fnd = jnp.where(hit, 1, fnd)
        ge = cnt >= _K
        return (it + 1, jnp.where(ge, lo, mid), jnp.where(ge, mid, hi),
                tf, fnd)

    _, _, hi, tf, fnd = jax.lax.while_loop(
        cond, step, (jnp.int32(0), lo0, hi0, tf0, fnd0))
    thr = jnp.where(fnd == 1, tf, hi)
    mask = keys <= thr
    zeros = jnp.zeros_like(d)
    ox_ref[0] = jnp.where(mask, jnp.broadcast_to(px, d.shape), zeros)
    oy_ref[0] = jnp.where(mask, jnp.broadcast_to(py, d.shape), zeros)
    oz_ref[0] = jnp.where(mask, jnp.broadcast_to(pz, d.shape), zeros)


@jax.jit
def kernel(pointcloud, target):
    pc = pointcloud[..., :3]
    b, n, _ = pc.shape
    q = target.shape[1]
    pc_t = jnp.swapaxes(pc, 1, 2)
    plane = jax.ShapeDtypeStruct((b, q, n), jnp.float32)
    ox, oy, oz = pl.pallas_call(
        _body,
        grid=(b,),
        in_specs=[
            pl.BlockSpec((1, 3, n), lambda i: (i, 0, 0)),
            pl.BlockSpec((1, q, 3), lambda i: (i, 0, 0)),
        ],
        out_specs=[pl.BlockSpec((1, q, n), lambda i: (i, 0, 0))] * 3,
        out_shape=[plane] * 3,
    )(pc_t, target)
    return jnp.stack([ox, oy, oz], axis=-1)


# speedup vs baseline: 5.6176x; 1.1915x over previous
import jax
import jax.numpy as jnp
from jax.experimental import pallas as pl

_K = 64


def _body(pc_ref, tgt_ref, ox_ref, oy_ref, oz_ref):
    pc = pc_ref[0]
    tgt = tgt_ref[0]
    px = pc[0:1, :]
    py = pc[1:2, :]
    pz = pc[2:3, :]
    npc = px * px + py * py + pz * pz
    nt = jnp.sum(tgt * tgt, axis=1, keepdims=True)
    def r16(x):
        return x.astype(jnp.bfloat16).astype(jnp.float32)
    dot = (r16(tgt[:, 0:1]) * r16(px) + r16(tgt[:, 1:2]) * r16(py)
           + r16(tgt[:, 2:3]) * r16(pz))
    d = nt + npc - 2.0 * dot

    s = jax.lax.bitcast_convert_type(d, jnp.int32)
    int_min = jnp.int32(-(2**31))
    keys = jnp.where(s >= 0, s, jnp.bitwise_xor(jnp.bitwise_not(s), int_min))

    q = keys.shape[0]
    lo0 = jnp.full((q, 1), -(2**31), jnp.int32)
    hi0 = jnp.full((q, 1), 2**31 - 1, jnp.int32)
    tf0 = jnp.full((q, 1), 2**31 - 1, jnp.int32)
    fnd0 = jnp.zeros((q, 1), jnp.int32)

    def cond(st):
        it, lo, hi, tf, fnd = st
        return jnp.logical_and(it < 32, jnp.min(fnd) == 0)

    def step(st):
        it, lo, hi, tf, fnd = st
        for _ in range(4):
            mid = (lo >> 1) + (hi >> 1) + (lo & hi & 1)
            cnt = jnp.sum((keys <= mid).astype(jnp.int32), axis=1,
                          keepdims=True)
            hit = jnp.logical_and(cnt == _K, fnd == 0)
            tf = jnp.where(hit, mid, tf)
            fnd = jnp.where(hit, 1, fnd)
            ge = cnt >= _K
            lo = jnp.where(ge, lo, mid)
            hi = jnp.where(ge, mid, hi)
        return (it + 4, lo, hi, tf, fnd)

    _, _, hi, tf, fnd = jax.lax.while_loop(
        cond, step, (jnp.int32(0), lo0, hi0, tf0, fnd0))
    thr = jnp.where(fnd == 1, tf, hi)
    mask = keys <= thr
    zeros = jnp.zeros_like(d)
    ox_ref[0] = jnp.where(mask, jnp.broadcast_to(px, d.shape), zeros)
    oy_ref[0] = jnp.where(mask, jnp.broadcast_to(py, d.shape), zeros)
    oz_ref[0] = jnp.where(mask, jnp.broadcast_to(pz, d.shape), zeros)


@jax.jit
def kernel(pointcloud, target):
    pc = pointcloud[..., :3]
    b, n, _ = pc.shape
    q = target.shape[1]
    pc_t = jnp.swapaxes(pc, 1, 2)
    plane = jax.ShapeDtypeStruct((b, q, n), jnp.float32)
    ox, oy, oz = pl.pallas_call(
        _body,
        grid=(b,),
        in_specs=[
            pl.BlockSpec((1, 3, n), lambda i: (i, 0, 0)),
            pl.BlockSpec((1, q, 3), lambda i: (i, 0, 0)),
        ],
        out_specs=[pl.BlockSpec((1, q, n), lambda i: (i, 0, 0))] * 3,
        out_shape=[plane] * 3,
    )(pc_t, target)
    return jnp.stack([ox, oy, oz], axis=-1)
